# baseline (speedup 1.0000x reference)
import functools

import jax
import jax.numpy as jnp
from jax import lax
from jax.experimental import pallas as pl
from jax.experimental.pallas import tpu as pltpu
from jax.experimental.pallas import tpu_sc as plsc

N = 10000
NP = 10240
E = 160000
NC = 2
NS = 16
NW = NC * NS
EPT = E // NW
K = 40
NCHUNK = EPT // K
RB = 5
NROUND = NCHUNK // RB
NB = 4
RPT = NP // NS

_MESH = plsc.VectorSubcoreMesh(core_axis_name="c", subcore_axis_name="s")


def _make_deg_kernel():
  DDEEP = 4

  @functools.partial(
      pl.kernel,
      out_type=jax.ShapeDtypeStruct((NC, NP), jnp.float32),
      mesh=_MESH,
      compiler_params=pltpu.CompilerParams(use_tc_tiling_on_sc=False),
      scratch_types=[
          pltpu.VMEM((NCHUNK, K), jnp.int32),
          pltpu.VMEM((K,), jnp.float32),
          pltpu.VMEM_SHARED((NP,), jnp.float32),
          pltpu.SemaphoreType.DMA,
      ],
  )
  def deg_kernel(col_hbm, ones_hbm, zeros_hbm, out_hbm, col_v, ones_v, dacc,
                 sem):
    c = lax.axis_index("c")
    s = lax.axis_index("s")
    t = s * NC + c
    base = s * RPT
    pltpu.sync_copy(zeros_hbm.at[pl.ds(base, RPT)], dacc.at[pl.ds(base, RPT)])
    pltpu.sync_copy(col_hbm.at[t], col_v)
    pltpu.sync_copy(ones_hbm, ones_v)
    plsc.subcore_barrier()

    for j in range(DDEEP):
      pltpu.async_copy(ones_v, dacc.at[col_v.at[j]], sem, add=True)

    def body(j, carry):
      @pl.when(j + DDEEP < NCHUNK)
      def _():
        pltpu.async_copy(ones_v, dacc.at[col_v.at[j + DDEEP]], sem, add=True)

      pltpu.make_async_copy(ones_v, dacc.at[col_v.at[j]], sem).wait()
      return carry

    lax.fori_loop(0, NCHUNK, body, 0)
    plsc.subcore_barrier()
    pltpu.sync_copy(dacc.at[pl.ds(base, RPT)],
                    out_hbm.at[c].at[pl.ds(base, RPT)])

  return deg_kernel


def _make_agg_kernel(D, GD, SD):
  NBUF = GD + SD

  @functools.partial(
      pl.kernel,
      out_type=jax.ShapeDtypeStruct((NC, NP, D), jnp.float32),
      mesh=_MESH,
      compiler_params=pltpu.CompilerParams(use_tc_tiling_on_sc=False),
      scratch_types=[
          pltpu.VMEM((NCHUNK, K), jnp.int32),
          pltpu.VMEM((NCHUNK, K), jnp.int32),
          pltpu.VMEM((NBUF * K, D), jnp.float32),
          pltpu.VMEM_SHARED((NP, D), jnp.float32),
          pltpu.SemaphoreType.DMA,
          pltpu.SemaphoreType.DMA,
      ],
  )
  def agg_kernel(row_hbm, col_hbm, src_hbm, zeros_hbm, out_hbm,
                 row_v, col_v, buf, acc, gsem, ssem):
    c = lax.axis_index("c")
    s = lax.axis_index("s")
    t = s * NC + c
    base = s * RPT

    @pl.when(c == 0)
    def _():
      pltpu.sync_copy(src_hbm.at[pl.ds(base, RPT)], acc.at[pl.ds(base, RPT)])

    @pl.when(c != 0)
    def _():
      pltpu.sync_copy(zeros_hbm.at[pl.ds(base, RPT)], acc.at[pl.ds(base, RPT)])

    pltpu.sync_copy(row_hbm.at[t], row_v)
    pltpu.sync_copy(col_hbm.at[t], col_v)
    plsc.subcore_barrier()

    def slot(j):
      return (j % NBUF) * K

    def gissue(j):
      pltpu.async_copy(src_hbm.at[row_v.at[j]],
                       buf.at[pl.ds(slot(j), K)], gsem)

    def gwait(j):
      pltpu.make_async_copy(src_hbm.at[row_v.at[j]],
                            buf.at[pl.ds(slot(j), K)], gsem).wait()

    def sissue(j):
      pltpu.async_copy(buf.at[pl.ds(slot(j), K)],
                       acc.at[col_v.at[j]], ssem, add=True)

    def swait(j):
      pltpu.make_async_copy(buf.at[pl.ds(slot(j), K)],
                            acc.at[col_v.at[j]], ssem).wait()

    for j in range(GD):
      gissue(j)

    def body(j, carry):
      gwait(j)
      sissue(j)

      @pl.when(j + GD < NCHUNK)
      def _():
        @pl.when(j >= SD)
        def _():
          swait(j - SD)

        gissue(j + GD)

      return carry

    lax.fori_loop(0, NCHUNK, body, 0)
    for j in range(max(0, NCHUNK - NBUF), NCHUNK):
      swait(j)
    plsc.subcore_barrier()
    pltpu.sync_copy(acc.at[pl.ds(base, RPT)],
                    out_hbm.at[c].at[pl.ds(base, RPT)])

  return agg_kernel


_deg_kernel = _make_deg_kernel()
_agg128 = _make_agg_kernel(128, 4, 3)
_agg40 = _make_agg_kernel(40, 5, 5)


_RBLK = 640
_NBLK = NP // _RBLK


def _prescale_body(deg_ref, x_ref, xs_ref, dinv_ref):
  deg = deg_ref[:, 0:1] + deg_ref[:, 1:2] + 1.0
  dinv = lax.rsqrt(deg)
  dinv_ref[...] = dinv
  xs_ref[...] = x_ref[...] * dinv


_PBLK = 400


def _tc_prescale(deg_t, x):
  return pl.pallas_call(
      _prescale_body,
      grid=(N // _PBLK,),
      in_specs=[
          pl.BlockSpec((_PBLK, NC), lambda i: (i, 0)),
          pl.BlockSpec((_PBLK, 128), lambda i: (i, 0)),
      ],
      out_specs=[
          pl.BlockSpec((_PBLK, 128), lambda i: (i, 0)),
          pl.BlockSpec((_PBLK, 1), lambda i: (i, 0)),
      ],
      out_shape=[
          jax.ShapeDtypeStruct((NP, 128), jnp.float32),
          jax.ShapeDtypeStruct((NP, 1), jnp.float32),
      ],
  )(deg_t, x)


def _mm_body(p_ref, dinv_ref, w1_ref, b1_ref, w2_ref, ys_ref):
  dinv = dinv_ref[...]
  a = (p_ref[0] + p_ref[1]) * dinv
  h = jnp.dot(a.astype(jnp.bfloat16), w1_ref[...].astype(jnp.bfloat16),
              preferred_element_type=jnp.float32)
  h = jnp.maximum(h + b1_ref[...], 0.0)
  y = jnp.dot(h.astype(jnp.bfloat16), w2_ref[...].astype(jnp.bfloat16),
              preferred_element_type=jnp.float32)
  ys_ref[...] = y * dinv


_MBLK = 1280


def _tc_mm(p, dinv, w1, b1, w2):
  return pl.pallas_call(
      _mm_body,
      grid=(NP // _MBLK,),
      in_specs=[
          pl.BlockSpec((NC, _MBLK, 128), lambda i: (0, i, 0)),
          pl.BlockSpec((_MBLK, 1), lambda i: (i, 0)),
          pl.BlockSpec((128, 2048), lambda i: (0, 0)),
          pl.BlockSpec((1, 2048), lambda i: (0, 0)),
          pl.BlockSpec((2048, 40), lambda i: (0, 0)),
      ],
      out_specs=pl.BlockSpec((_MBLK, 40), lambda i: (i, 0)),
      out_shape=jax.ShapeDtypeStruct((NP, 40), jnp.float32),
  )(p, dinv, w1, b1, w2)


def _final_body(q_ref, dinv_ref, b2_ref, out_ref):
  out_ref[...] = (q_ref[0] + q_ref[1]) * dinv_ref[...] + b2_ref[...]


def _tc_final(q, dinv, b2p):
  return pl.pallas_call(
      _final_body,
      grid=(N // _PBLK,),
      in_specs=[
          pl.BlockSpec((NC, _PBLK, 40), lambda i: (0, i, 0)),
          pl.BlockSpec((_PBLK, 1), lambda i: (i, 0)),
          pl.BlockSpec((1, 40), lambda i: (0, 0)),
      ],
      out_specs=pl.BlockSpec((_PBLK, 40), lambda i: (i, 0)),
      out_shape=jax.ShapeDtypeStruct((N, 40), jnp.float32),
  )(q, dinv, b2p)


def kernel(x, edge_index, W1, b1, W2, b2):
  ei = edge_index.astype(jnp.int32)
  row2 = ei[0].reshape(NW, NCHUNK, K)
  col2 = ei[1].reshape(NW, NCHUNK, K)
  b1r = b1.reshape(1, 2048)
  b2r = b2.reshape(1, 40)
  ones_k = jnp.ones((K,), jnp.float32)
  z1 = jnp.zeros((NP,), jnp.float32)
  z128 = jnp.zeros((NP, 128), jnp.float32)
  z40 = jnp.zeros((NP, 40), jnp.float32)

  degp = _deg_kernel(col2, ones_k, z1)
  deg_t = degp.T
  xs, dinv = _tc_prescale(deg_t, x)
  p = _agg128(row2, col2, xs, z128)
  ys = _tc_mm(p, dinv, W1, b1r, W2)
  q = _agg40(row2, col2, ys, z40)
  return _tc_final(q, dinv, b2r)


# speedup vs baseline: 1.0298x; 1.0298x over previous
import functools

import jax
import jax.numpy as jnp
from jax import lax
from jax.experimental import pallas as pl
from jax.experimental.pallas import tpu as pltpu
from jax.experimental.pallas import tpu_sc as plsc

N = 10000
NP = 10240
E = 160000
NC = 2
NS = 16
NW = NC * NS
EPT = E // NW
K = 40
NCHUNK = EPT // K
RB = 5
NROUND = NCHUNK // RB
NB = 4
RPT = NP // NS

_MESH = plsc.VectorSubcoreMesh(core_axis_name="c", subcore_axis_name="s")


def _make_deg_kernel():
  DDEEP = 4

  @functools.partial(
      pl.kernel,
      out_type=jax.ShapeDtypeStruct((NC, NP), jnp.float32),
      mesh=_MESH,
      compiler_params=pltpu.CompilerParams(use_tc_tiling_on_sc=False),
      scratch_types=[
          pltpu.VMEM((NCHUNK, K), jnp.int32),
          pltpu.VMEM((K,), jnp.float32),
          pltpu.VMEM_SHARED((NP,), jnp.float32),
          pltpu.SemaphoreType.DMA,
      ],
  )
  def deg_kernel(col_hbm, ones_hbm, zeros_hbm, out_hbm, col_v, ones_v, dacc,
                 sem):
    c = lax.axis_index("c")
    s = lax.axis_index("s")
    t = s * NC + c
    base = s * RPT
    pltpu.sync_copy(zeros_hbm.at[pl.ds(base, RPT)], dacc.at[pl.ds(base, RPT)])
    pltpu.sync_copy(col_hbm.at[t], col_v)
    pltpu.sync_copy(ones_hbm, ones_v)
    plsc.subcore_barrier()

    for j in range(DDEEP):
      pltpu.async_copy(ones_v, dacc.at[col_v.at[j]], sem, add=True)

    def body(j, carry):
      @pl.when(j + DDEEP < NCHUNK)
      def _():
        pltpu.async_copy(ones_v, dacc.at[col_v.at[j + DDEEP]], sem, add=True)

      pltpu.make_async_copy(ones_v, dacc.at[col_v.at[j]], sem).wait()
      return carry

    lax.fori_loop(0, NCHUNK, body, 0)
    plsc.subcore_barrier()
    pltpu.sync_copy(dacc.at[pl.ds(base, RPT)],
                    out_hbm.at[c].at[pl.ds(base, RPT)])

  return deg_kernel


def _make_agg_kernel(D, GD, SD):
  NBUF = GD + SD

  @functools.partial(
      pl.kernel,
      out_type=jax.ShapeDtypeStruct((NC, NP, D), jnp.float32),
      mesh=_MESH,
      compiler_params=pltpu.CompilerParams(use_tc_tiling_on_sc=False),
      scratch_types=[
          pltpu.VMEM((NCHUNK, K), jnp.int32),
          pltpu.VMEM((NCHUNK, K), jnp.int32),
          pltpu.VMEM((NBUF * K, D), jnp.float32),
          pltpu.VMEM_SHARED((NP, D), jnp.float32),
          pltpu.SemaphoreType.DMA,
          pltpu.SemaphoreType.DMA,
      ],
  )
  def agg_kernel(row_hbm, col_hbm, src_hbm, zeros_hbm, out_hbm,
                 row_v, col_v, buf, acc, gsem, ssem):
    c = lax.axis_index("c")
    s = lax.axis_index("s")
    t = s * NC + c
    base = s * RPT

    @pl.when(c == 0)
    def _():
      pltpu.sync_copy(src_hbm.at[pl.ds(base, RPT)], acc.at[pl.ds(base, RPT)])

    @pl.when(c != 0)
    def _():
      pltpu.sync_copy(zeros_hbm.at[pl.ds(base, RPT)], acc.at[pl.ds(base, RPT)])

    pltpu.sync_copy(row_hbm.at[t], row_v)
    pltpu.sync_copy(col_hbm.at[t], col_v)
    plsc.subcore_barrier()

    def slot(j):
      return (j % NBUF) * K

    def gissue(j):
      pltpu.async_copy(src_hbm.at[row_v.at[j]],
                       buf.at[pl.ds(slot(j), K)], gsem)

    def gwait(j):
      pltpu.make_async_copy(src_hbm.at[row_v.at[j]],
                            buf.at[pl.ds(slot(j), K)], gsem).wait()

    def sissue(j):
      pltpu.async_copy(buf.at[pl.ds(slot(j), K)],
                       acc.at[col_v.at[j]], ssem, add=True)

    def swait(j):
      pltpu.make_async_copy(buf.at[pl.ds(slot(j), K)],
                            acc.at[col_v.at[j]], ssem).wait()

    for j in range(GD):
      gissue(j)

    def body(j, carry):
      gwait(j)
      sissue(j)

      @pl.when(j + GD < NCHUNK)
      def _():
        @pl.when(j >= SD)
        def _():
          swait(j - SD)

        gissue(j + GD)

      return carry

    lax.fori_loop(0, NCHUNK, body, 0)
    for j in range(max(0, NCHUNK - NBUF), NCHUNK):
      swait(j)
    plsc.subcore_barrier()
    pltpu.sync_copy(acc.at[pl.ds(base, RPT)],
                    out_hbm.at[c].at[pl.ds(base, RPT)])

  return agg_kernel


_deg_kernel = _make_deg_kernel()
_agg128 = _make_agg_kernel(128, 4, 3)
_agg40 = _make_agg_kernel(40, 5, 5)


_RBLK = 640
_NBLK = NP // _RBLK


def _prescale_body(deg_ref, x_ref, xs_ref, dinv_ref):
  deg = deg_ref[:, 0:1] + deg_ref[:, 1:2] + 1.0
  dinv = lax.rsqrt(deg)
  dinv_ref[...] = dinv
  xs_ref[...] = x_ref[...] * dinv


def _tc_prescale(deg_t, x_pad):
  return pl.pallas_call(
      _prescale_body,
      grid=(_NBLK,),
      in_specs=[
          pl.BlockSpec((_RBLK, NC), lambda i: (i, 0)),
          pl.BlockSpec((_RBLK, 128), lambda i: (i, 0)),
      ],
      out_specs=[
          pl.BlockSpec((_RBLK, 128), lambda i: (i, 0)),
          pl.BlockSpec((_RBLK, 1), lambda i: (i, 0)),
      ],
      out_shape=[
          jax.ShapeDtypeStruct((NP, 128), jnp.float32),
          jax.ShapeDtypeStruct((NP, 1), jnp.float32),
      ],
  )(deg_t, x_pad)


def _mm_body(p_ref, dinv_ref, w1_ref, b1_ref, w2_ref, ys_ref):
  dinv = dinv_ref[...]
  a = (p_ref[0] + p_ref[1]) * dinv
  h = jnp.dot(a.astype(jnp.bfloat16), w1_ref[...].astype(jnp.bfloat16),
              preferred_element_type=jnp.float32)
  h = jnp.maximum(h + b1_ref[...], 0.0)
  y = jnp.dot(h.astype(jnp.bfloat16), w2_ref[...].astype(jnp.bfloat16),
              preferred_element_type=jnp.float32)
  ys_ref[...] = y * dinv


_MBLK = 1280


def _tc_mm(p, dinv, w1, b1, w2):
  return pl.pallas_call(
      _mm_body,
      grid=(NP // _MBLK,),
      in_specs=[
          pl.BlockSpec((NC, _MBLK, 128), lambda i: (0, i, 0)),
          pl.BlockSpec((_MBLK, 1), lambda i: (i, 0)),
          pl.BlockSpec((128, 2048), lambda i: (0, 0)),
          pl.BlockSpec((1, 2048), lambda i: (0, 0)),
          pl.BlockSpec((2048, 40), lambda i: (0, 0)),
      ],
      out_specs=pl.BlockSpec((_MBLK, 40), lambda i: (i, 0)),
      out_shape=jax.ShapeDtypeStruct((NP, 40), jnp.float32),
  )(p, dinv, w1, b1, w2)


def _final_body(q_ref, dinv_ref, b2_ref, out_ref):
  out_ref[...] = (q_ref[0] + q_ref[1]) * dinv_ref[...] + b2_ref[...]


def _tc_final(q, dinv, b2p):
  return pl.pallas_call(
      _final_body,
      grid=(_NBLK,),
      in_specs=[
          pl.BlockSpec((NC, _RBLK, 40), lambda i: (0, i, 0)),
          pl.BlockSpec((_RBLK, 1), lambda i: (i, 0)),
          pl.BlockSpec((1, 40), lambda i: (0, 0)),
      ],
      out_specs=pl.BlockSpec((_RBLK, 40), lambda i: (i, 0)),
      out_shape=jax.ShapeDtypeStruct((NP, 40), jnp.float32),
  )(q, dinv, b2p)


def kernel(x, edge_index, W1, b1, W2, b2):
  ei = edge_index.astype(jnp.int32)
  row2 = ei[0].reshape(NW, NCHUNK, K)
  col2 = ei[1].reshape(NW, NCHUNK, K)
  x_pad = jnp.pad(x, ((0, NP - N), (0, 0)))
  b1r = b1.reshape(1, 2048)
  b2r = b2.reshape(1, 40)
  ones_k = jnp.ones((K,), jnp.float32)
  z1 = jnp.zeros((NP,), jnp.float32)
  z128 = jnp.zeros((NP, 128), jnp.float32)
  z40 = jnp.zeros((NP, 40), jnp.float32)

  degp = _deg_kernel(col2, ones_k, z1)
  deg_t = degp.T
  xs, dinv = _tc_prescale(deg_t, x_pad)
  p = _agg128(row2, col2, xs, z128)
  ys = _tc_mm(p, dinv, W1, b1r, W2)
  q = _agg40(row2, col2, ys, z40)
  outp = _tc_final(q, dinv, b2r)
  return outp[:N]
